# baseline (speedup 1.0000x reference)
import functools

import jax
import jax.numpy as jnp
from jax import lax
from jax.experimental import pallas as pl
from jax.experimental.pallas import tpu as pltpu
from jax.experimental.pallas import tpu_sc as plsc

_EPS = 1e-5


_B = 128


def _build_segment_partials(N, D, E):
    NC, NS = 2, 16
    NW = NC * NS
    B = _B
    assert E % B == 0
    NBT = E // B
    NB = NBT // NW
    EXTRA = NBT - NB * NW
    assert NB >= 4 and (NB - 2) % 4 == 0
    RPS = (N // (NS * 8)) * 8
    TAIL = N - RPS * NS
    assert TAIL % 8 == 0 and 0 <= TAIL < NS * 8

    mesh = plsc.VectorSubcoreMesh(core_axis_name="c", subcore_axis_name="s")

    @functools.partial(
        pl.kernel,
        out_type=(
            jax.ShapeDtypeStruct((N, D), jnp.float32),
            jax.ShapeDtypeStruct((N, D), jnp.float32),
        ),
        mesh=mesh,
        scratch_types=[
            pltpu.VMEM((4, B), jnp.int32),
            pltpu.VMEM((4, B), jnp.int32),
            pltpu.VMEM((2, B, D), jnp.float32),
            pltpu.VMEM_SHARED((N, D), jnp.float32),
            pltpu.SemaphoreType.DMA,
            pltpu.SemaphoreType.DMA,
            pltpu.SemaphoreType.DMA,
            pltpu.SemaphoreType.DMA,
            pltpu.SemaphoreType.DMA,
            pltpu.SemaphoreType.DMA,
            pltpu.SemaphoreType.DMA,
            pltpu.SemaphoreType.DMA,
        ],
    )
    def seg_kernel(h_hbm, src_hbm, dst_hbm, zeros_hbm, out0, out1,
                   idx_s, idx_d, rows, acc, gsem0, gsem1, ssem0, ssem1,
                   isem0, isem1, isem2, isem3):
        cid = lax.axis_index("c")
        sid = lax.axis_index("s")
        wid = cid * NS + sid

        rbase = pl.multiple_of(sid * RPS, 8)
        pltpu.sync_copy(zeros_hbm.at[pl.ds(rbase, RPS)],
                        acc.at[pl.ds(rbase, RPS)])
        if TAIL:
            @pl.when(sid == 0)
            def _():
                pltpu.sync_copy(zeros_hbm.at[pl.ds(NS * RPS, TAIL)],
                                acc.at[pl.ds(NS * RPS, TAIL)])
        plsc.subcore_barrier()

        gsems = (gsem0, gsem1)
        ssems = (ssem0, ssem1)
        isems = (isem0, isem1, isem2, isem3)

        def idx_load(j, si):
            base = pl.multiple_of((j * NW + wid) * B, 8)
            pltpu.async_copy(src_hbm.at[pl.ds(base, B)], idx_s.at[si],
                             isems[si])
            pltpu.async_copy(dst_hbm.at[pl.ds(base, B)], idx_d.at[si],
                             isems[si])

        def idx_wait(si):
            pltpu.make_async_copy(src_hbm.at[pl.ds(0, B)], idx_s.at[si],
                                  isems[si]).wait()
            pltpu.make_async_copy(dst_hbm.at[pl.ds(0, B)], idx_d.at[si],
                                  isems[si]).wait()

        def gather(si, sr):
            pltpu.async_copy(h_hbm.at[idx_s.at[si]], rows.at[sr], gsems[sr])

        def gather_wait(si, sr):
            pltpu.make_async_copy(h_hbm.at[idx_s.at[si]], rows.at[sr],
                                  gsems[sr]).wait()

        def scatter(si, sr):
            pltpu.async_copy(rows.at[sr], acc.at[idx_d.at[si]], ssems[sr],
                             add=True)

        def scatter_wait(si, sr):
            pltpu.make_async_copy(rows.at[sr], acc.at[idx_d.at[si]],
                                  ssems[sr]).wait()

        def step(j, si, sr, with_ws, load_j2):
            idx_wait(si)
            if with_ws:
                scatter_wait(si, sr)
            gather(si, sr)
            gather_wait((si + 3) % 4, 1 - sr)
            scatter((si + 3) % 4, 1 - sr)
            if load_j2:
                idx_load(j + 2, (si + 2) % 4)

        idx_load(0, 0)
        idx_load(1, 1)
        idx_load(2, 2)
        idx_wait(0)
        gather(0, 0)
        step(1, 1, 1, False, True)

        NTAIL = ((NB - 2) % 4) + 4
        NQ = (NB - 2 - NTAIL) // 4

        def quad_body(q, carry):
            for u in range(4):
                j = 4 * q + 2 + u
                step(j, (2 + u) % 4, u % 2, True, True)
            return carry
        lax.fori_loop(0, NQ, quad_body, 0)
        for u in range(NTAIL):
            j = NB - NTAIL + u
            step(j, j % 4, j % 2, True, j + 2 < NB)
        gather_wait((NB - 1) % 4, (NB - 1) % 2)
        scatter((NB - 1) % 4, (NB - 1) % 2)
        scatter_wait((NB - 2) % 4, (NB - 2) % 2)
        scatter_wait((NB - 1) % 4, (NB - 1) % 2)

        if EXTRA:
            @pl.when(wid < EXTRA)
            def _():
                idx_load(NB, 0)
                idx_wait(0)
                gather(0, 0)
                gather_wait(0, 0)
                scatter(0, 0)
                scatter_wait(0, 0)
        plsc.subcore_barrier()

        @pl.when(cid == 0)
        def _():
            pltpu.sync_copy(acc.at[pl.ds(rbase, RPS)], out0.at[pl.ds(rbase, RPS)])
            if TAIL:
                @pl.when(sid == 0)
                def _():
                    pltpu.sync_copy(acc.at[pl.ds(NS * RPS, TAIL)],
                                    out0.at[pl.ds(NS * RPS, TAIL)])

        @pl.when(cid == 1)
        def _():
            pltpu.sync_copy(acc.at[pl.ds(rbase, RPS)], out1.at[pl.ds(rbase, RPS)])
            if TAIL:
                @pl.when(sid == 0)
                def _():
                    pltpu.sync_copy(acc.at[pl.ds(NS * RPS, TAIL)],
                                    out1.at[pl.ds(NS * RPS, TAIL)])

    return seg_kernel


_R = 1000


def _dot(a, b):
    return jax.lax.dot_general(
        a, b, (((1,), (0,)), ((), ())),
        preferred_element_type=jnp.float32)


def _accum_stats(i, z, st_ref, acc_ref):
    blk = jnp.concatenate(
        [jnp.sum(z, axis=0)[None], jnp.sum(z * z, axis=0)[None]], axis=0)

    @pl.when(i == 0)
    def _():
        acc_ref[...] = blk

    @pl.when(i > 0)
    def _():
        acc_ref[...] = acc_ref[...] + blk

    @pl.when(i == pl.num_programs(0) - 1)
    def _():
        st_ref[...] = acc_ref[...]


def _bn_coeffs(st, gamma, beta, n):
    mean = st[0] / n
    var = st[1] / n - mean * mean
    inv = gamma[0] / jnp.sqrt(var + _EPS)
    shift = beta[0] - mean * inv
    return inv, shift


def _stage_a(h, p0, p1, wT):
    N, D = h.shape
    G = N // _R

    def body(x_ref, p0_ref, p1_ref, w_ref, z_ref, st_ref, acc_ref):
        i = pl.program_id(0)
        rst = x_ref[...] + p0_ref[...] + p1_ref[...]
        z = _dot(rst, w_ref[...])
        z_ref[...] = z
        _accum_stats(i, z, st_ref, acc_ref)

    row_spec = pl.BlockSpec((_R, D), lambda i: (i, 0))
    full_spec = pl.BlockSpec((D, D), lambda i: (0, 0))
    st_spec = pl.BlockSpec((2, D), lambda i: (0, 0))
    return pl.pallas_call(
        body,
        grid=(G,),
        in_specs=[row_spec, row_spec, row_spec, full_spec],
        out_specs=(row_spec, st_spec),
        out_shape=(jax.ShapeDtypeStruct((N, D), jnp.float32),
                   jax.ShapeDtypeStruct((2, D), jnp.float32)),
        scratch_shapes=[pltpu.VMEM((2, D), jnp.float32)],
    )(h, p0, p1, wT)


def _stage_b(z1, st1, gamma, beta, wT):
    N, D = z1.shape
    G = N // _R

    def body(x_ref, s_ref, g_ref, b_ref, w_ref, z_ref, st_ref, acc_ref):
        i = pl.program_id(0)
        inv, shift = _bn_coeffs(s_ref[...], g_ref[...], b_ref[...], N)
        u = jnp.maximum(x_ref[...] * inv[None] + shift[None], 0.0)
        z = _dot(u, w_ref[...])
        z_ref[...] = z
        _accum_stats(i, z, st_ref, acc_ref)

    row_spec = pl.BlockSpec((_R, D), lambda i: (i, 0))
    st_spec = pl.BlockSpec((2, D), lambda i: (0, 0))
    vec_spec = pl.BlockSpec((1, D), lambda i: (0, 0))
    full_spec = pl.BlockSpec((D, D), lambda i: (0, 0))
    return pl.pallas_call(
        body,
        grid=(G,),
        in_specs=[row_spec, st_spec, vec_spec, vec_spec, full_spec],
        out_specs=(row_spec, st_spec),
        out_shape=(jax.ShapeDtypeStruct((N, D), jnp.float32),
                   jax.ShapeDtypeStruct((2, D), jnp.float32)),
        scratch_shapes=[pltpu.VMEM((2, D), jnp.float32)],
    )(z1, st1, gamma, beta, wT)


def _stage_c(z2, st2, gamma, beta):
    N, D = z2.shape
    G = N // _R

    def body(x_ref, s_ref, g_ref, b_ref, v_ref, st_ref, acc_ref):
        i = pl.program_id(0)
        inv, shift = _bn_coeffs(s_ref[...], g_ref[...], b_ref[...], N)
        v = jnp.maximum(x_ref[...] * inv[None] + shift[None], 0.0)
        v_ref[...] = v
        _accum_stats(i, v, st_ref, acc_ref)

    row_spec = pl.BlockSpec((_R, D), lambda i: (i, 0))
    st_spec = pl.BlockSpec((2, D), lambda i: (0, 0))
    vec_spec = pl.BlockSpec((1, D), lambda i: (0, 0))
    return pl.pallas_call(
        body,
        grid=(G,),
        in_specs=[row_spec, st_spec, vec_spec, vec_spec],
        out_specs=(row_spec, st_spec),
        out_shape=(jax.ShapeDtypeStruct((N, D), jnp.float32),
                   jax.ShapeDtypeStruct((2, D), jnp.float32)),
        scratch_shapes=[pltpu.VMEM((2, D), jnp.float32)],
    )(z2, st2, gamma, beta)


def _stage_d(v, st3, gamma, beta, relu):
    N, D = v.shape
    G = N // _R

    def body(x_ref, s_ref, g_ref, b_ref, o_ref):
        inv, shift = _bn_coeffs(s_ref[...], g_ref[...], b_ref[...], N)
        z = x_ref[...] * inv[None] + shift[None]
        if relu:
            z = jnp.maximum(z, 0.0)
        o_ref[...] = z

    row_spec = pl.BlockSpec((_R, D), lambda i: (i, 0))
    st_spec = pl.BlockSpec((2, D), lambda i: (0, 0))
    vec_spec = pl.BlockSpec((1, D), lambda i: (0, 0))
    return pl.pallas_call(
        body,
        grid=(G,),
        in_specs=[row_spec, st_spec, vec_spec, vec_spec],
        out_specs=row_spec,
        out_shape=jax.ShapeDtypeStruct((N, D), jnp.float32),
    )(v, st3, gamma, beta)


def kernel(h, edge_index, W1, W2, mlp_bn_gamma, mlp_bn_beta,
           apply_bn_gamma, apply_bn_beta, out_bn_gamma, out_bn_beta):
    N, D = h.shape
    E = edge_index.shape[1]
    L = W1.shape[0]
    assert E % _B == 0
    src = edge_index[0]
    dst = edge_index[1]
    zeros = jnp.zeros((N, D), jnp.float32)
    seg = _build_segment_partials(N, D, E)

    for i in range(L):
        p0, p1 = seg(h, src, dst, zeros)
        z1, s1 = _stage_a(h, p0, p1, W1[i].T)
        z2, s2 = _stage_b(z1, s1, mlp_bn_gamma[i][None], mlp_bn_beta[i][None],
                          W2[i].T)
        v, s3 = _stage_c(z2, s2, apply_bn_gamma[i][None], apply_bn_beta[i][None])
        h = _stage_d(v, s3, out_bn_gamma[i][None], out_bn_beta[i][None],
                     relu=(i < L - 1))
    return h


# speedup vs baseline: 1.1139x; 1.1139x over previous
import functools

import jax
import jax.numpy as jnp
from jax import lax
from jax.experimental import pallas as pl
from jax.experimental.pallas import tpu as pltpu
from jax.experimental.pallas import tpu_sc as plsc

_EPS = 1e-5


_B = 128


def _build_segment_partials(N, D, E):
    NC, NS = 2, 16
    NW = NC * NS
    B = _B
    assert E % B == 0
    NBT = E // B
    NB = NBT // NW
    EXTRA = NBT - NB * NW
    assert NB >= 4 and (NB - 2) % 4 == 0
    RPS = (N // (NS * 8)) * 8
    TAIL = N - RPS * NS
    assert TAIL % 8 == 0 and 0 <= TAIL < NS * 8

    mesh = plsc.VectorSubcoreMesh(core_axis_name="c", subcore_axis_name="s")

    @functools.partial(
        pl.kernel,
        out_type=(
            jax.ShapeDtypeStruct((N, D), jnp.float32),
            jax.ShapeDtypeStruct((N, D), jnp.float32),
        ),
        mesh=mesh,
        scratch_types=[
            pltpu.VMEM((4, B), jnp.int32),
            pltpu.VMEM((4, B), jnp.int32),
            pltpu.VMEM((2, B, D), jnp.float32),
            pltpu.VMEM_SHARED((N, D), jnp.float32),
            pltpu.SemaphoreType.DMA,
            pltpu.SemaphoreType.DMA,
            pltpu.SemaphoreType.DMA,
            pltpu.SemaphoreType.DMA,
            pltpu.SemaphoreType.DMA,
            pltpu.SemaphoreType.DMA,
            pltpu.SemaphoreType.DMA,
            pltpu.SemaphoreType.DMA,
        ],
    )
    def seg_kernel(h_hbm, src_hbm, dst_hbm, zeros_hbm, out0, out1,
                   idx_s, idx_d, rows, acc, gsem0, gsem1, ssem0, ssem1,
                   isem0, isem1, isem2, isem3):
        cid = lax.axis_index("c")
        sid = lax.axis_index("s")
        wid = cid * NS + sid

        rbase = pl.multiple_of(sid * RPS, 8)
        pltpu.sync_copy(zeros_hbm.at[pl.ds(rbase, RPS)],
                        acc.at[pl.ds(rbase, RPS)])
        if TAIL:
            @pl.when(sid == 0)
            def _():
                pltpu.sync_copy(zeros_hbm.at[pl.ds(NS * RPS, TAIL)],
                                acc.at[pl.ds(NS * RPS, TAIL)])
        plsc.subcore_barrier()

        gsems = (gsem0, gsem1)
        ssems = (ssem0, ssem1)
        isems = (isem0, isem1, isem2, isem3)

        def idx_load(j, si):
            base = pl.multiple_of((j * NW + wid) * B, 8)
            pltpu.async_copy(src_hbm.at[pl.ds(base, B)], idx_s.at[si],
                             isems[si])
            pltpu.async_copy(dst_hbm.at[pl.ds(base, B)], idx_d.at[si],
                             isems[si])

        def idx_wait(si):
            pltpu.make_async_copy(src_hbm.at[pl.ds(0, B)], idx_s.at[si],
                                  isems[si]).wait()
            pltpu.make_async_copy(dst_hbm.at[pl.ds(0, B)], idx_d.at[si],
                                  isems[si]).wait()

        def gather(si, sr):
            pltpu.async_copy(h_hbm.at[idx_s.at[si]], rows.at[sr], gsems[sr])

        def gather_wait(si, sr):
            pltpu.make_async_copy(h_hbm.at[idx_s.at[si]], rows.at[sr],
                                  gsems[sr]).wait()

        def scatter(si, sr):
            pltpu.async_copy(rows.at[sr], acc.at[idx_d.at[si]], ssems[sr],
                             add=True)

        def scatter_wait(si, sr):
            pltpu.make_async_copy(rows.at[sr], acc.at[idx_d.at[si]],
                                  ssems[sr]).wait()

        def step(j, si, sr, with_ws, load_j2):
            idx_wait(si)
            if with_ws:
                scatter_wait(si, sr)
            gather(si, sr)
            gather_wait((si + 3) % 4, 1 - sr)
            scatter((si + 3) % 4, 1 - sr)
            if load_j2:
                idx_load(j + 2, (si + 2) % 4)

        idx_load(0, 0)
        idx_load(1, 1)
        idx_load(2, 2)
        idx_wait(0)
        gather(0, 0)
        step(1, 1, 1, False, True)

        NTAIL = ((NB - 2) % 4) + 4
        NQ = (NB - 2 - NTAIL) // 4

        def quad_body(q, carry):
            for u in range(4):
                j = 4 * q + 2 + u
                step(j, (2 + u) % 4, u % 2, True, True)
            return carry
        lax.fori_loop(0, NQ, quad_body, 0)
        for u in range(NTAIL):
            j = NB - NTAIL + u
            step(j, j % 4, j % 2, True, j + 2 < NB)
        gather_wait((NB - 1) % 4, (NB - 1) % 2)
        scatter((NB - 1) % 4, (NB - 1) % 2)
        scatter_wait((NB - 2) % 4, (NB - 2) % 2)
        scatter_wait((NB - 1) % 4, (NB - 1) % 2)

        if EXTRA:
            @pl.when(wid < EXTRA)
            def _():
                idx_load(NB, 0)
                idx_wait(0)
                gather(0, 0)
                gather_wait(0, 0)
                scatter(0, 0)
                scatter_wait(0, 0)
        plsc.subcore_barrier()

        @pl.when(cid == 0)
        def _():
            pltpu.sync_copy(acc.at[pl.ds(rbase, RPS)], out0.at[pl.ds(rbase, RPS)])
            if TAIL:
                @pl.when(sid == 0)
                def _():
                    pltpu.sync_copy(acc.at[pl.ds(NS * RPS, TAIL)],
                                    out0.at[pl.ds(NS * RPS, TAIL)])

        @pl.when(cid == 1)
        def _():
            pltpu.sync_copy(acc.at[pl.ds(rbase, RPS)], out1.at[pl.ds(rbase, RPS)])
            if TAIL:
                @pl.when(sid == 0)
                def _():
                    pltpu.sync_copy(acc.at[pl.ds(NS * RPS, TAIL)],
                                    out1.at[pl.ds(NS * RPS, TAIL)])

    return seg_kernel


_R = 1000


def _dot(a, b):
    return jax.lax.dot_general(
        a, b, (((1,), (0,)), ((), ())),
        preferred_element_type=jnp.float32)


def _accum_stats(i, z, st_ref, acc_ref):
    blk = jnp.concatenate(
        [jnp.sum(z, axis=0)[None], jnp.sum(z * z, axis=0)[None]], axis=0)

    @pl.when(i == 0)
    def _():
        acc_ref[...] = blk

    @pl.when(i > 0)
    def _():
        acc_ref[...] = acc_ref[...] + blk

    @pl.when(i == pl.num_programs(0) - 1)
    def _():
        st_ref[...] = acc_ref[...]


def _bn_coeffs(st, gamma, beta, n):
    mean = st[0] / n
    var = st[1] / n - mean * mean
    inv = gamma[0] / jnp.sqrt(var + _EPS)
    shift = beta[0] - mean * inv
    return inv, shift


def _dense_layer(h, p0, p1, w1T, w2T, g1, b1, ga, ba, go, bo, relu_last):
    N, D = h.shape
    G = N // _R

    def body(h_ref, p0_ref, p1_ref, w1_ref, w2_ref, g1_ref, b1_ref,
             ga_ref, ba_ref, go_ref, bo_ref, o_ref, zbuf, s1, s2, s3):
        p = pl.program_id(0)
        i = pl.program_id(1)
        rs = pl.ds(i * _R, _R)

        def accum(z, acc_ref):
            blk = jnp.concatenate(
                [jnp.sum(z, axis=0)[None], jnp.sum(z * z, axis=0)[None]],
                axis=0)

            @pl.when(i == 0)
            def _():
                acc_ref[...] = blk

            @pl.when(i > 0)
            def _():
                acc_ref[...] = acc_ref[...] + blk

        @pl.when(p == 0)
        def _():
            rst = h_ref[...] + p0_ref[...] + p1_ref[...]
            z = _dot(rst, w1_ref[...])
            zbuf[rs, :] = z
            accum(z, s1)

        @pl.when(p == 1)
        def _():
            inv, shift = _bn_coeffs(s1[...], g1_ref[...], b1_ref[...], N)
            u = jnp.maximum(zbuf[rs, :] * inv[None] + shift[None], 0.0)
            z = _dot(u, w2_ref[...])
            zbuf[rs, :] = z
            accum(z, s2)

        @pl.when(p == 2)
        def _():
            inv, shift = _bn_coeffs(s2[...], ga_ref[...], ba_ref[...], N)
            v = jnp.maximum(zbuf[rs, :] * inv[None] + shift[None], 0.0)
            zbuf[rs, :] = v
            accum(v, s3)

        @pl.when(p == 3)
        def _():
            inv, shift = _bn_coeffs(s3[...], go_ref[...], bo_ref[...], N)
            z = zbuf[rs, :] * inv[None] + shift[None]
            if relu_last:
                z = jnp.maximum(z, 0.0)
            o_ref[...] = z

    first_spec = pl.BlockSpec((_R, D), lambda p, i: (i * (p == 0), 0))
    last_spec = pl.BlockSpec((_R, D), lambda p, i: (i * (p == 3), 0))
    full_spec = pl.BlockSpec((D, D), lambda p, i: (0, 0))
    vec_spec = pl.BlockSpec((1, D), lambda p, i: (0, 0))
    return pl.pallas_call(
        body,
        grid=(4, G),
        in_specs=[first_spec, first_spec, first_spec, full_spec, full_spec,
                  vec_spec, vec_spec, vec_spec, vec_spec, vec_spec, vec_spec],
        out_specs=last_spec,
        out_shape=jax.ShapeDtypeStruct((N, D), jnp.float32),
        scratch_shapes=[pltpu.VMEM((N, D), jnp.float32),
                        pltpu.VMEM((2, D), jnp.float32),
                        pltpu.VMEM((2, D), jnp.float32),
                        pltpu.VMEM((2, D), jnp.float32)],
    )(h, p0, p1, w1T, w2T, g1, b1, ga, ba, go, bo)


def kernel(h, edge_index, W1, W2, mlp_bn_gamma, mlp_bn_beta,
           apply_bn_gamma, apply_bn_beta, out_bn_gamma, out_bn_beta):
    N, D = h.shape
    E = edge_index.shape[1]
    L = W1.shape[0]
    assert E % _B == 0
    src = edge_index[0]
    dst = edge_index[1]
    zeros = jnp.zeros((N, D), jnp.float32)
    seg = _build_segment_partials(N, D, E)

    for i in range(L):
        p0, p1 = seg(h, src, dst, zeros)
        h = _dense_layer(h, p0, p1, W1[i].T, W2[i].T,
                         mlp_bn_gamma[i][None], mlp_bn_beta[i][None],
                         apply_bn_gamma[i][None], apply_bn_beta[i][None],
                         out_bn_gamma[i][None], out_bn_beta[i][None],
                         relu_last=(i < L - 1))
    return h
